# Initial kernel scaffold; baseline (speedup 1.0000x reference)
import jax
import jax.numpy as jnp
from jax.experimental import pallas as pl


def kernel(x, pos_emb):
    raise NotImplementedError("write your pallas kernel here")


# speedup vs baseline: 1.7303x; 1.7303x over previous
import jax
import jax.numpy as jnp
from jax.experimental import pallas as pl


_TN = 512


def _add_block(x_ref, pe_ref, o_ref):
    o_ref[...] = x_ref[...] + pe_ref[...]


def kernel(x, pos_emb):
    B, N, D = x.shape
    pe = pos_emb[:N]
    return pl.pallas_call(
        _add_block,
        grid=(N // _TN,),
        in_specs=[
            pl.BlockSpec((B, _TN, D), lambda n: (0, n, 0)),
            pl.BlockSpec((_TN, D), lambda n: (n, 0)),
        ],
        out_specs=pl.BlockSpec((B, _TN, D), lambda n: (0, n, 0)),
        out_shape=jax.ShapeDtypeStruct((B, N, D), x.dtype),
    )(x, pe)
